# Initial kernel scaffold; baseline (speedup 1.0000x reference)
import jax
import jax.numpy as jnp
from jax.experimental import pallas as pl


def kernel(image, labels, weights):
    raise NotImplementedError("write your pallas kernel here")


# speedup vs baseline: 27.1805x; 27.1805x over previous
import functools

import jax
import jax.numpy as jnp
from jax import lax
from jax.experimental import pallas as pl
from jax.experimental.pallas import tpu as pltpu
from jax.experimental.pallas import tpu_sc as plsc

PS = 64
K = 64
TEMP = 0.3
NOISE = 0.5
STRIDE = 16
GRID = 29

NC = 2
NS = 16
NW = NC * NS

PATCHES_PER_TILE = (16 * K) // NW
IMG_ROWS = 3 * PS * 4
LBL_ROWS = 1 * PS * 4
IMG_CHUNKS = IMG_ROWS // 128
LBL_CHUNKS = LBL_ROWS // 128


def _sc_patch_gather(image, labels, harr, warr):
    mesh = plsc.VectorSubcoreMesh(
        core_axis_name="c", subcore_axis_name="s",
        num_cores=NC, num_subcores=NS)

    ppt = PATCHES_PER_TILE

    @functools.partial(
        pl.kernel,
        out_type=[
            jax.ShapeDtypeStruct((16 * K, 3, PS, PS), jnp.float32),
            jax.ShapeDtypeStruct((16 * K, 1, PS, PS), jnp.float32),
        ],
        mesh=mesh,
        scratch_types=[
            pltpu.VMEM((ppt,), jnp.int32),
            pltpu.VMEM((ppt,), jnp.int32),
            pltpu.VMEM((3, PS, PS), jnp.float32),
            pltpu.VMEM((1, PS, PS), jnp.float32),
            pltpu.SemaphoreType.DMA,
        ],
        compiler_params=pltpu.CompilerParams(use_tc_tiling_on_sc=False),
    )
    def k(img_hbm, lbl_hbm, h_hbm, w_hbm, out_img, out_lbl,
          h_v, w_v, ibuf, lbuf, sem):
        wid = lax.axis_index("s") * NC + lax.axis_index("c")
        pltpu.sync_copy(h_hbm.at[pl.ds(wid * ppt, ppt)], h_v)
        pltpu.sync_copy(w_hbm.at[pl.ds(wid * ppt, ppt)], w_v)

        def chunk_body(cidx, carry):
            hv = h_v[pl.ds(cidx * 16, 16)]
            wv = w_v[pl.ds(cidx * 16, 16)]
            for lane in range(16):
                h = pl.multiple_of(hv[lane], 16)
                w = pl.multiple_of(wv[lane], 16)
                patch = wid * ppt + cidx * 16 + lane
                b = patch // K
                descs = []
                for c in range(3):
                    descs.append(pltpu.async_copy(
                        img_hbm.at[b, c, pl.ds(h, PS), pl.ds(w, PS)],
                        ibuf.at[c], sem))
                descs.append(pltpu.async_copy(
                    lbl_hbm.at[b, 0, pl.ds(h, PS), pl.ds(w, PS)],
                    lbuf.at[0], sem))
                for d in descs:
                    d.wait()
                pltpu.sync_copy(ibuf, out_img.at[patch])
                pltpu.sync_copy(lbuf, out_lbl.at[patch])
            return carry

        lax.fori_loop(0, ppt // 16, chunk_body, 0)

    return k(image, labels, harr, warr)


def kernel(image, labels, weights):
    B, C, H, W = image.shape
    ps = PS

    s = lax.reduce_window(weights, 0.0, lax.add,
                          (1, 1, ps, ps), (1, 1, STRIDE, STRIDE), 'VALID')
    scores = (s / float(ps * ps)).reshape(B, GRID * GRID)
    smin = jnp.min(scores, axis=1, keepdims=True)
    smax = jnp.max(scores, axis=1, keepdims=True)
    srange = jnp.clip(smax - smin, 1e-06, None)
    normalized = (scores - smin) / srange
    scaled = normalized / TEMP
    nkey = jax.random.fold_in(jax.random.key(0), 1)
    scaled = scaled + jax.random.uniform(nkey, scaled.shape, dtype=scaled.dtype) * NOISE
    probs = jax.nn.softmax(scaled, axis=1)
    gkey = jax.random.fold_in(jax.random.key(0), 2)
    gumbel = jax.random.gumbel(gkey, probs.shape, dtype=probs.dtype)
    _, idx = jax.lax.top_k(jnp.log(probs + 1e-20) + gumbel, K)
    row = idx // GRID
    col = idx % GRID
    h = jnp.clip(row * STRIDE, 0, H - ps).astype(jnp.int32)
    w = jnp.clip(col * STRIDE, 0, W - ps).astype(jnp.int32)
    coords = jnp.stack([h, w], axis=-1)

    out_img, out_lbl = _sc_patch_gather(
        image, labels, h.reshape(B * K), w.reshape(B * K))
    patches = out_img.reshape(B, K, C, ps, ps)
    patch_labels = out_lbl.reshape(B, K, 1, ps, ps)
    return patches, patch_labels, coords
